# baseline (speedup 1.0000x reference)
import jax
import jax.numpy as jnp
from jax.experimental import pallas as pl
from jax.experimental.pallas import tpu as pltpu


def _se_nhwc_kernel(x_ref, w1s_ref, w2_ref, y_ref):
    x = x_ref[...]
    pooled = jnp.sum(x, axis=1)
    h = jnp.dot(pooled, w1s_ref[...], preferred_element_type=jnp.float32)
    h = jnp.maximum(h, 0.0)
    g = jnp.dot(h, w2_ref[...], preferred_element_type=jnp.float32)
    gate = jax.nn.sigmoid(g)
    y_ref[...] = x * gate[:, None, :]


def _batch_tile(B, row_bytes, budget):
    tb = max(1, budget // (4 * row_bytes))
    while B % tb != 0:
        tb -= 1
    return tb


def kernel(x_nchw, w1_t, w2_t):
    B, C, H, W = x_nchw.shape
    HW = H * W
    Cr = w1_t.shape[1]

    x_pix = jnp.transpose(x_nchw, (0, 2, 3, 1)).reshape(B, HW, C)

    w1s = w1_t.astype(jnp.float32) * jnp.float32(1.0 / HW)
    w2f = w2_t.astype(jnp.float32)

    row_bytes = C * HW * x_nchw.dtype.itemsize
    tb = _batch_tile(B, row_bytes, 24 * 1024 * 1024)
    tb = 2

    y_pix = pl.pallas_call(
        _se_nhwc_kernel,
        out_shape=jax.ShapeDtypeStruct((B, HW, C), x_nchw.dtype),
        grid=(B // tb,),
        in_specs=[
            pl.BlockSpec((tb, HW, C), lambda b: (b, 0, 0)),
            pl.BlockSpec((C, Cr), lambda b: (0, 0)),
            pl.BlockSpec((Cr, C), lambda b: (0, 0)),
        ],
        out_specs=pl.BlockSpec((tb, HW, C), lambda b: (b, 0, 0)),
        compiler_params=pltpu.CompilerParams(
            dimension_semantics=("arbitrary",),
            vmem_limit_bytes=48 * 1024 * 1024,
        ),
    )(x_pix, w1s, w2f)

    return jnp.transpose(y_pix.reshape(B, H, W, C), (0, 3, 1, 2))


# speedup vs baseline: 1.2129x; 1.2129x over previous
import jax
import jax.numpy as jnp
from jax.experimental import pallas as pl
from jax.experimental.pallas import tpu as pltpu


def _se_nhwc_kernel(x_ref, w1s_ref, w2_ref, y_ref):
    x = x_ref[...]
    pooled = jnp.sum(x, axis=1)
    h = jnp.dot(pooled, w1s_ref[...], preferred_element_type=jnp.float32)
    h = jnp.maximum(h, 0.0)
    g = jnp.dot(h, w2_ref[...], preferred_element_type=jnp.float32)
    gate = jax.nn.sigmoid(g)
    y_ref[...] = x * gate[:, None, :]


def _batch_tile(B, row_bytes, budget):
    tb = max(1, budget // (4 * row_bytes))
    while B % tb != 0:
        tb -= 1
    return tb


def kernel(x_nchw, w1_t, w2_t):
    B, C, H, W = x_nchw.shape
    HW = H * W
    Cr = w1_t.shape[1]

    x_pix = jnp.transpose(x_nchw, (0, 2, 3, 1)).reshape(B, HW, C)

    w1s = w1_t.astype(jnp.float32) * jnp.float32(1.0 / HW)
    w2f = w2_t.astype(jnp.float32)

    row_bytes = C * HW * x_nchw.dtype.itemsize
    tb = _batch_tile(B, row_bytes, 24 * 1024 * 1024)
    tb = 8

    y_pix = pl.pallas_call(
        _se_nhwc_kernel,
        out_shape=jax.ShapeDtypeStruct((B, HW, C), x_nchw.dtype),
        grid=(B // tb,),
        in_specs=[
            pl.BlockSpec((tb, HW, C), lambda b: (b, 0, 0)),
            pl.BlockSpec((C, Cr), lambda b: (0, 0)),
            pl.BlockSpec((Cr, C), lambda b: (0, 0)),
        ],
        out_specs=pl.BlockSpec((tb, HW, C), lambda b: (b, 0, 0)),
        compiler_params=pltpu.CompilerParams(
            dimension_semantics=("arbitrary",),
            vmem_limit_bytes=48 * 1024 * 1024,
        ),
    )(x_pix, w1s, w2f)

    return jnp.transpose(y_pix.reshape(B, H, W, C), (0, 3, 1, 2))


# speedup vs baseline: 1.2149x; 1.0016x over previous
import jax
import jax.numpy as jnp
from jax.experimental import pallas as pl
from jax.experimental.pallas import tpu as pltpu


def _se_nhwc_kernel(x_ref, w1s_ref, w2_ref, y_ref):
    x = x_ref[...]
    pooled = jnp.sum(x, axis=1)
    h = jnp.dot(pooled, w1s_ref[...], preferred_element_type=jnp.float32)
    h = jnp.maximum(h, 0.0)
    g = jnp.dot(h, w2_ref[...], preferred_element_type=jnp.float32)
    gate = jax.nn.sigmoid(g)
    y_ref[...] = x * gate[:, None, :]


def _batch_tile(B, row_bytes, budget):
    tb = max(1, min(B, budget // (4 * row_bytes)))
    while B % tb != 0:
        tb -= 1
    return tb


def kernel(x_nchw, w1_t, w2_t):
    B, C, H, W = x_nchw.shape
    HW = H * W
    Cr = w1_t.shape[1]

    x_pix = jnp.transpose(x_nchw, (0, 2, 3, 1)).reshape(B, HW, C)

    w1s = w1_t.astype(jnp.float32) * jnp.float32(1.0 / HW)
    w2f = w2_t.astype(jnp.float32)

    row_bytes = C * HW * x_nchw.dtype.itemsize
    tb = _batch_tile(B, row_bytes, 32 * 1024 * 1024)

    y_pix = pl.pallas_call(
        _se_nhwc_kernel,
        out_shape=jax.ShapeDtypeStruct((B, HW, C), x_nchw.dtype),
        grid=(B // tb,),
        in_specs=[
            pl.BlockSpec((tb, HW, C), lambda b: (b, 0, 0)),
            pl.BlockSpec((C, Cr), lambda b: (0, 0)),
            pl.BlockSpec((Cr, C), lambda b: (0, 0)),
        ],
        out_specs=pl.BlockSpec((tb, HW, C), lambda b: (b, 0, 0)),
        compiler_params=pltpu.CompilerParams(
            dimension_semantics=("arbitrary",),
            vmem_limit_bytes=48 * 1024 * 1024,
        ),
    )(x_pix, w1s, w2f)

    return jnp.transpose(y_pix.reshape(B, H, W, C), (0, 3, 1, 2))


# speedup vs baseline: 1.2157x; 1.0007x over previous
import jax
import jax.numpy as jnp
from jax.experimental import pallas as pl
from jax.experimental.pallas import tpu as pltpu


def _se_nhwc_kernel(x_ref, w1s_ref, w2_ref, y_ref):
    x = x_ref[...]
    pooled = jnp.sum(x, axis=1)
    h = jnp.dot(pooled, w1s_ref[...], preferred_element_type=jnp.float32)
    h = jnp.maximum(h, 0.0)
    g = jnp.dot(h, w2_ref[...], preferred_element_type=jnp.float32)
    gate = jax.nn.sigmoid(g)
    y_ref[...] = x * gate[:, None, :]


def _batch_tile(B, row_bytes, budget):
    tb = max(1, min(B, budget // (4 * row_bytes)))
    while B % tb != 0:
        tb -= 1
    return tb


def kernel(x_nchw, w1_t, w2_t):
    B, C, H, W = x_nchw.shape
    HW = H * W
    Cr = w1_t.shape[1]

    x_pix = jnp.transpose(x_nchw, (0, 2, 3, 1)).reshape(B, HW, C)

    w1s = w1_t.astype(jnp.float32) * jnp.float32(1.0 / HW)
    w2f = w2_t.astype(jnp.float32)

    row_bytes = C * HW * x_nchw.dtype.itemsize
    tb = _batch_tile(B, row_bytes, 32 * 1024 * 1024)

    y_pix = pl.pallas_call(
        _se_nhwc_kernel,
        out_shape=jax.ShapeDtypeStruct((B, HW, C), x_nchw.dtype),
        grid=(B // tb,),
        in_specs=[
            pl.BlockSpec((tb, HW, C), lambda b: (b, 0, 0)),
            pl.BlockSpec((C, Cr), lambda b: (0, 0)),
            pl.BlockSpec((Cr, C), lambda b: (0, 0)),
        ],
        out_specs=pl.BlockSpec((tb, HW, C), lambda b: (b, 0, 0)),
        compiler_params=pltpu.CompilerParams(
            dimension_semantics=("parallel",),
            vmem_limit_bytes=48 * 1024 * 1024,
        ),
    )(x_pix, w1s, w2f)

    return jnp.transpose(y_pix.reshape(B, H, W, C), (0, 3, 1, 2))
